# baseline (speedup 1.0000x reference)
import functools

import jax
import jax.numpy as jnp
from jax import lax
from jax.experimental import pallas as pl
from jax.experimental.pallas import tpu as pltpu
from jax.experimental.pallas import tpu_sc as plsc

N = 10000
E = 320000
F_IN = 128
H = 16
K = 3000
NC = 2

NP = 10240
NW = 32
EB = 128
NB = 80
GRP = 8
PADE = NW * NB * EB
ROWS_PER_TILE = NP // 16
R = NP // 8


def _zero_ref(ref, n16):
    def _zero(i, carry):
        ref[pl.ds(i * 16, 16)] = jnp.zeros((16,), jnp.float32)
        return carry

    lax.fori_loop(0, n16, _zero, 0)


def _seg16_body(table_hbm, srcp_hbm, dstp_hbm, out_hbm, src2, dst2, rows,
                zbuf, acc_sh, gsem, ssem):
    c = lax.axis_index("c")
    s = lax.axis_index("s")
    wid = c * 16 + s

    pltpu.sync_copy(srcp_hbm.at[wid], src2)
    pltpu.sync_copy(dstp_hbm.at[wid], dst2)

    def _zero(i, carry):
        zbuf[i, :] = jnp.zeros((16,), jnp.float32)
        return carry

    lax.fori_loop(0, ROWS_PER_TILE, _zero, 0)
    pltpu.sync_copy(zbuf, acc_sh.at[pl.ds(s * ROWS_PER_TILE, ROWS_PER_TILE)])
    plsc.subcore_barrier()

    ngroups = NB // GRP

    def fire_gathers(g, buf):
        return [pltpu.async_copy(
            table_hbm.at[src2.at[g * GRP + b]], rows.at[buf, b], gsem)
            for b in range(GRP)]

    def fire_scatters(g, buf):
        return [pltpu.async_copy(
            rows.at[buf, b], acc_sh.at[dst2.at[g * GRP + b]], ssem, add=True)
            for b in range(GRP)]

    depth = 3
    descs_g = [None] * ngroups
    descs_s = [None] * ngroups
    for g in range(min(depth - 1, ngroups)):
        descs_g[g] = fire_gathers(g, g % depth)
    for g in range(ngroups):
        buf = g % depth
        nxt = g + depth - 1
        if nxt < ngroups:
            prev_s = nxt - depth
            if prev_s >= 0:
                for d in descs_s[prev_s]:
                    d.wait()
            descs_g[nxt] = fire_gathers(nxt, nxt % depth)
        for d in descs_g[g]:
            d.wait()
        descs_s[g] = fire_scatters(g, buf)
    for g in range(max(0, ngroups - depth), ngroups):
        if descs_s[g] is not None:
            for d in descs_s[g]:
                d.wait()

    plsc.subcore_barrier()
    off = c * NP + s * ROWS_PER_TILE
    pltpu.sync_copy(acc_sh.at[pl.ds(s * ROWS_PER_TILE, ROWS_PER_TILE)],
                    out_hbm.at[pl.ds(off, ROWS_PER_TILE)])


def _deg1d_body(dstp_hbm, out_hbm, dst2, ones_row, zbuf, acc_sh, ssem):
    c = lax.axis_index("c")
    s = lax.axis_index("s")
    wid = c * 16 + s

    pltpu.sync_copy(dstp_hbm.at[wid], dst2)

    def _ones(i, carry):
        ones_row[pl.ds(i * 16, 16)] = jnp.ones((16,), jnp.float32)
        return carry

    lax.fori_loop(0, EB // 16, _ones, 0)
    _zero_ref(zbuf, ROWS_PER_TILE // 16)
    pltpu.sync_copy(zbuf, acc_sh.at[pl.ds(s * ROWS_PER_TILE, ROWS_PER_TILE)])
    plsc.subcore_barrier()

    descs = [pltpu.async_copy(ones_row, acc_sh.at[dst2.at[b]], ssem,
                              add=True)
             for b in range(NB)]
    for d in descs:
        d.wait()

    plsc.subcore_barrier()
    off = c * NP + s * ROWS_PER_TILE
    pltpu.sync_copy(acc_sh.at[pl.ds(s * ROWS_PER_TILE, ROWS_PER_TILE)],
                    out_hbm.at[pl.ds(off, ROWS_PER_TILE)])


def _seg1d_body(table_hbm, srcp_hbm, dstp_hbm, out_hbm, src2, dst2, rows,
                zbuf, acc_sh, gsem, ssem):
    c = lax.axis_index("c")
    s = lax.axis_index("s")
    wid = c * 16 + s

    pltpu.sync_copy(srcp_hbm.at[wid], src2)
    pltpu.sync_copy(dstp_hbm.at[wid], dst2)
    _zero_ref(zbuf, ROWS_PER_TILE // 16)
    pltpu.sync_copy(zbuf, acc_sh.at[pl.ds(s * ROWS_PER_TILE, ROWS_PER_TILE)])
    plsc.subcore_barrier()

    ngroups = NB // GRP

    def fire_gathers(g, buf):
        return [pltpu.async_copy(
            table_hbm.at[src2.at[g * GRP + b]], rows.at[buf, b], gsem)
            for b in range(GRP)]

    def fire_scatters(g, buf):
        return [pltpu.async_copy(
            rows.at[buf, b], acc_sh.at[dst2.at[g * GRP + b]], ssem, add=True)
            for b in range(GRP)]

    depth = 3
    descs_g = [None] * ngroups
    descs_s = [None] * ngroups
    for g in range(min(depth - 1, ngroups)):
        descs_g[g] = fire_gathers(g, g % depth)
    for g in range(ngroups):
        buf = g % depth
        nxt = g + depth - 1
        if nxt < ngroups:
            prev_s = nxt - depth
            if prev_s >= 0:
                for d in descs_s[prev_s]:
                    d.wait()
            descs_g[nxt] = fire_gathers(nxt, nxt % depth)
        for d in descs_g[g]:
            d.wait()
        descs_s[g] = fire_scatters(g, buf)
    for g in range(max(0, ngroups - depth), ngroups):
        if descs_s[g] is not None:
            for d in descs_s[g]:
                d.wait()

    plsc.subcore_barrier()
    off = c * NP + s * ROWS_PER_TILE
    pltpu.sync_copy(acc_sh.at[pl.ds(s * ROWS_PER_TILE, ROWS_PER_TILE)],
                    out_hbm.at[pl.ds(off, ROWS_PER_TILE)])


_SC_MESH = plsc.VectorSubcoreMesh(core_axis_name="c", subcore_axis_name="s")
_SC_PARAMS = pltpu.CompilerParams(use_tc_tiling_on_sc=False)

_seg16 = functools.partial(
    pl.kernel,
    out_type=jax.ShapeDtypeStruct((2 * NP, H), jnp.float32),
    mesh=_SC_MESH,
    compiler_params=_SC_PARAMS,
    scratch_types=[
        pltpu.VMEM((NB, EB), jnp.int32),
        pltpu.VMEM((NB, EB), jnp.int32),
        pltpu.VMEM((3, GRP, EB, H), jnp.float32),
        pltpu.VMEM((ROWS_PER_TILE, H), jnp.float32),
        pltpu.VMEM_SHARED((NP, H), jnp.float32),
        pltpu.SemaphoreType.DMA,
        pltpu.SemaphoreType.DMA,
    ],
)(_seg16_body)

_deg1d = functools.partial(
    pl.kernel,
    out_type=jax.ShapeDtypeStruct((2 * NP,), jnp.float32),
    mesh=_SC_MESH,
    compiler_params=_SC_PARAMS,
    scratch_types=[
        pltpu.VMEM((NB, EB), jnp.int32),
        pltpu.VMEM((EB,), jnp.float32),
        pltpu.VMEM((ROWS_PER_TILE,), jnp.float32),
        pltpu.VMEM_SHARED((NP,), jnp.float32),
        pltpu.SemaphoreType.DMA,
    ],
)(_deg1d_body)

_seg1d = functools.partial(
    pl.kernel,
    out_type=jax.ShapeDtypeStruct((2 * NP,), jnp.float32),
    mesh=_SC_MESH,
    compiler_params=_SC_PARAMS,
    scratch_types=[
        pltpu.VMEM((NB, EB), jnp.int32),
        pltpu.VMEM((NB, EB), jnp.int32),
        pltpu.VMEM((3, GRP, EB), jnp.float32),
        pltpu.VMEM((ROWS_PER_TILE,), jnp.float32),
        pltpu.VMEM_SHARED((NP,), jnp.float32),
        pltpu.SemaphoreType.DMA,
        pltpu.SemaphoreType.DMA,
    ],
)(_seg1d_body)


def _iota2(shape, dim):
    return lax.broadcasted_iota(jnp.int32, shape, dim)


def _blockdiag(w, nb):
    bi, bo = w.shape
    t = jnp.tile(w, (nb, nb))
    sh = (nb * bi, nb * bo)
    return jnp.where(_iota2(sh, 0) // bi == _iota2(sh, 1) // bo, t, 0.0)


def _gt8():
    return jnp.where(_iota2((8, 128), 0) == _iota2((8, 128), 1) // H,
                     1.0, 0.0).astype(jnp.float32)


def _g8():
    return jnp.where(_iota2((128, 8), 0) // H == _iota2((128, 8), 1),
                     1.0, 0.0).astype(jnp.float32)


def _tca_body(xp_ref, w1_ref, out_ref):
    w1bd = _blockdiag(w1_ref[...], 8)
    out_ref[...] = jnp.dot(xp_ref[...], w1bd,
                           preferred_element_type=jnp.float32)


def _tcb_body(p0_ref, xw1_ref, xs1_ref, dinv_ref, invdeg_ref):
    deg8 = p0_ref[pl.ds(0, R), :] + p0_ref[pl.ds(R, R), :] + 1.0
    gt8 = _gt8()
    dinv = jnp.dot(lax.rsqrt(deg8), gt8, preferred_element_type=jnp.float32)
    invdeg = jnp.dot(1.0 / deg8, gt8, preferred_element_type=jnp.float32)
    dinv_ref[...] = dinv
    invdeg_ref[...] = invdeg
    xs1_ref[...] = xw1_ref[...] * dinv


def _tcc_body(p1_ref, xw1_ref, dinv_ref, invdeg_ref, b1_ref,
              h1_ref, t2_ref):
    accs = p1_ref[pl.ds(0, R), :] + p1_ref[pl.ds(R, R), :]
    b1t = jnp.tile(b1_ref[...], (1, 8))
    h1 = jax.nn.relu(dinv_ref[...] * accs + xw1_ref[...] * invdeg_ref[...]
                     + b1t)
    h1_ref[...] = h1
    t2_ref[...] = dinv_ref[...] * h1


def _radix_desc_select(u, limit, nbits, cmp_le):
    def level(i, t):
        k = (nbits - 2) - 2 * i
        cnts = []
        for j in (1, 2, 3):
            cand = t | (jnp.uint32(j) << k.astype(jnp.uint32)) \
                if u.dtype == jnp.uint32 else t | (j << k)
            if cmp_le:
                c = jnp.sum((u < cand).astype(jnp.int32))
                cnts.append((c <= limit).astype(jnp.uint32)
                            if u.dtype == jnp.uint32
                            else (c <= limit).astype(jnp.int32))
            else:
                c = jnp.sum((u >= cand).astype(jnp.int32))
                cnts.append((c >= limit).astype(jnp.uint32)
                            if u.dtype == jnp.uint32
                            else (c >= limit).astype(jnp.int32))
        nsel = cnts[0] + cnts[1] + cnts[2]
        if u.dtype == jnp.uint32:
            return t | (nsel << k.astype(jnp.uint32))
        return t | (nsel << k)

    zero = jnp.uint32(0) if u.dtype == jnp.uint32 else jnp.int32(0)
    return lax.fori_loop(0, nbits // 2, level, zero)


def _tcd_body(p2_ref, h1_ref, dinv_ref, invdeg_ref, wsc_ref,
              w2_ref, bsc_ref, xw2_ref, keptf8_ref):
    acch = p2_ref[pl.ds(0, R), :] + p2_ref[pl.ds(R, R), :]
    svec = dinv_ref[...] * acch + h1_ref[...] * invdeg_ref[...]
    wscbd = _blockdiag(wsc_ref[...], 8)
    score = jnp.dot(svec, wscbd, preferred_element_type=jnp.float32) \
        + bsc_ref[...]
    b = lax.bitcast_convert_type(score, jnp.int32)
    v = b ^ ((b >> 31) & jnp.int32(0x7FFFFFFF))
    u = lax.bitcast_convert_type(v ^ jnp.int32(-2147483648), jnp.uint32)
    node = _iota2((R, 8), 0) * 8 + _iota2((R, 8), 1)
    u = jnp.where(node < N, u, jnp.uint32(0))

    t = _radix_desc_select(u, K, 32, cmp_le=False)
    c_gt = jnp.sum((u > t).astype(jnp.int32))
    need = K - c_gt
    nodeq = jnp.where(u == t, node, jnp.int32(2 ** 14))
    p = _radix_desc_select(nodeq, need, 14, cmp_le=True)
    kept = (u > t) | ((u == t) & (node < p))
    keptf8 = kept.astype(jnp.float32)
    keptf8_ref[...] = keptf8
    kg8 = keptf8 * jnp.tanh(score)
    kg = jnp.dot(kg8, _gt8(), preferred_element_type=jnp.float32)
    xq = kg * h1_ref[...]
    w2bd = _blockdiag(w2_ref[...], 8)
    xw2_ref[...] = jnp.dot(xq, w2bd, preferred_element_type=jnp.float32)


def _tce_body(p3_ref, xw2_ref, xs2_ref, dinv2_ref, invdeg2_ref):
    deg8 = p3_ref[pl.ds(0, R), :] + p3_ref[pl.ds(R, R), :] + 1.0
    gt8 = _gt8()
    dinv2 = jnp.dot(lax.rsqrt(deg8), gt8, preferred_element_type=jnp.float32)
    invdeg2 = jnp.dot(1.0 / deg8, gt8, preferred_element_type=jnp.float32)
    dinv2_ref[...] = dinv2
    invdeg2_ref[...] = invdeg2
    xs2_ref[...] = xw2_ref[...] * dinv2


def _tcf_body(p4_ref, xw2_ref, dinv2_ref, invdeg2_ref, keptf8_ref,
              b2_ref, lng_ref, lnb_ref, gw1_ref, gb1_ref, bng_ref, bnb_ref,
              gw2_ref, gb2_ref, fw1_ref, fb1_ref, fw2_ref, fb2_ref,
              out_ref):
    acc2 = p4_ref[pl.ds(0, R), :] + p4_ref[pl.ds(R, R), :]
    b2t = jnp.tile(b2_ref[...], (1, 8))
    h2 = jax.nn.relu(dinv2_ref[...] * acc2 + xw2_ref[...] * invdeg2_ref[...]
                     + b2t)
    g8 = _g8()
    gt8 = _gt8()
    mu8 = jnp.dot(h2, g8, preferred_element_type=jnp.float32) * (1.0 / H)
    mu = jnp.dot(mu8, gt8, preferred_element_type=jnp.float32)
    d = h2 - mu
    var8 = jnp.dot(d * d, g8, preferred_element_type=jnp.float32) * (1.0 / H)
    inv8 = lax.rsqrt(var8 + 1e-5)
    inv = jnp.dot(inv8, gt8, preferred_element_type=jnp.float32)
    hn = d * inv * jnp.tile(lng_ref[...], (1, 8)) \
        + jnp.tile(lnb_ref[...], (1, 8))
    kf8 = keptf8_ref[...]
    f64t = jnp.where(_iota2((8, 64), 0) == _iota2((8, 64), 1) % 8,
                     1.0, 0.0).astype(jnp.float32)
    f64 = jnp.where(_iota2((64, 8), 0) % 8 == _iota2((64, 8), 1),
                    1.0, 0.0).astype(jnp.float32)
    gw1bd = _blockdiag(gw1_ref[...], 8)
    g = jnp.dot(hn, gw1bd, preferred_element_type=jnp.float32) \
        + jnp.tile(gb1_ref[...], (1, 8))
    kf64 = jnp.dot(kf8, f64t, preferred_element_type=jnp.float32)
    msum = jnp.sum(kf64 * g, axis=0, keepdims=True)
    bmu8 = jnp.dot(msum, f64, preferred_element_type=jnp.float32) \
        * (1.0 / K)
    bmu = jnp.dot(bmu8, f64t, preferred_element_type=jnp.float32)
    dg = g - bmu
    vsum = jnp.sum(kf64 * dg * dg, axis=0, keepdims=True)
    bvar8 = jnp.dot(vsum, f64, preferred_element_type=jnp.float32) \
        * (1.0 / K)
    bvar = jnp.dot(bvar8, f64t, preferred_element_type=jnp.float32)
    gn = dg / jnp.sqrt(bvar + 1e-5) * jnp.tile(bng_ref[...], (1, 8)) \
        + jnp.tile(bnb_ref[...], (1, 8))
    gn = jax.nn.relu(gn)
    gw2bd = _blockdiag(gw2_ref[...], 8)
    gsc8 = jnp.dot(gn, gw2bd, preferred_element_type=jnp.float32) \
        + gb2_ref[...]
    m = jnp.max(jnp.where(kf8 > 0.0, gsc8, -jnp.inf))
    a8 = kf8 * jnp.exp(gsc8 - m)
    z = jnp.sum(a8)
    aexp = jnp.dot(a8, gt8, preferred_element_type=jnp.float32)
    pooled128 = jnp.sum(aexp * hn, axis=0, keepdims=True) * (1.0 / z)
    shf = (128, H)
    fold = jnp.where(_iota2(shf, 0) % H == _iota2(shf, 1),
                     1.0, 0.0).astype(jnp.float32)
    pooled = jnp.dot(pooled128, fold, preferred_element_type=jnp.float32)
    h = jnp.tanh(jnp.dot(pooled, fw1_ref[...],
                         preferred_element_type=jnp.float32) + fb1_ref[...])
    logits = jnp.dot(h, fw2_ref[...], preferred_element_type=jnp.float32) \
        + fb2_ref[...]
    logits = logits - jnp.max(logits, axis=-1, keepdims=True)
    ez = jnp.exp(logits)
    out_ref[...] = ez / jnp.sum(ez, axis=-1, keepdims=True)


def _call(body, out_shapes, *args):
    return pl.pallas_call(body, out_shape=out_shapes)(*args)


def kernel(x_origin, edge_index, edge_weight, pos, params):
    f32 = jnp.float32
    sd = jax.ShapeDtypeStruct
    src = edge_index[0].astype(jnp.int32)
    dst = edge_index[1].astype(jnp.int32)
    fill = N + (jnp.arange(PADE - E, dtype=jnp.int32) % (NP - N))
    srcp = jnp.concatenate([src, fill]).reshape(NW, NB, EB)
    dstp = jnp.concatenate([dst, fill]).reshape(NW, NB, EB)

    xp = jnp.pad(x_origin, ((0, NP - N), (0, 0))).reshape(R, 8 * F_IN)

    def seg(table_p):
        return _seg16(table_p.reshape(NP, H), srcp, dstp).reshape(2 * R, 128)

    p0 = _deg1d(dstp).reshape(2 * R, 8)
    xw1 = _call(_tca_body, sd((R, 128), f32), xp, params['W1'])
    xs1, dinv1, invdeg1 = _call(
        _tcb_body, [sd((R, 128), f32)] * 3, p0, xw1)
    p1 = seg(xs1)
    h1, t2 = _call(
        _tcc_body, [sd((R, 128), f32)] * 2,
        p1, xw1, dinv1, invdeg1, params['b1'].reshape(1, H))
    p2 = seg(t2)
    xw2, keptf8 = _call(
        _tcd_body, [sd((R, 128), f32), sd((R, 8), f32)],
        p2, h1, dinv1, invdeg1, params['Wsc'], params['W2'],
        params['bsc'].reshape(1, 1))
    p3 = _seg1d(keptf8.reshape(NP), srcp, dstp).reshape(2 * R, 8)
    xs2, dinv2, invdeg2 = _call(
        _tce_body, [sd((R, 128), f32)] * 3, p3, xw2)
    p4 = seg(xs2)
    out = _call(
        _tcf_body, sd((1, NC), f32),
        p4, xw2, dinv2, invdeg2, keptf8,
        params['b2'].reshape(1, H), params['ln_g'].reshape(1, H),
        params['ln_b'].reshape(1, H), params['gW1'],
        params['gb1'].reshape(1, H // 2), params['bn_g'].reshape(1, H // 2),
        params['bn_b'].reshape(1, H // 2), params['gW2'],
        params['gb2'].reshape(1, 1), params['fW1'],
        params['fb1'].reshape(1, H // 2), params['fW2'],
        params['fb2'].reshape(1, NC))
    return out
